# baseline (speedup 1.0000x reference)
import functools

import jax
import jax.numpy as jnp
from jax import lax
from jax.experimental import pallas as pl
from jax.experimental.pallas import tpu as pltpu
from jax.experimental.pallas import tpu_sc as plsc

NB = 16384
NC, NS = 2, 16
NW = NC * NS
BPW = NB // NW
CH = 128
NCH = BPW // CH
NJOB = 3 * NCH
NBUF = 7

DT, DG, DF = 64, 32, 16
PB = 4096
BS = 2048
GN = 2


def _proj2_body(tva_ref, tvb_ref, w_ref, out_ref):
    pa = lax.dot_general(tva_ref[...], w_ref[...],
                         (((0,), (0,)), ((), ())),
                         precision=lax.Precision.DEFAULT,
                         preferred_element_type=jnp.float32)
    pb = lax.dot_general(tvb_ref[...], w_ref[...],
                         (((0,), (0,)), ((), ())),
                         precision=lax.Precision.DEFAULT,
                         preferred_element_type=jnp.float32)
    out_ref[...] = jnp.concatenate([pa, pb], axis=-1)


def _project2(table_t, w):
    d, v = table_t.shape
    nblk = pl.cdiv(v, PB)
    npair = pl.cdiv(nblk, 2)
    last = nblk - 1
    return pl.pallas_call(
        _proj2_body,
        grid=(npair,),
        in_specs=[
            pl.BlockSpec((d, PB), lambda i: (0, 2 * i)),
            pl.BlockSpec((d, PB), lambda i: (0, jnp.minimum(2 * i + 1, last))),
            pl.BlockSpec((d, 64), lambda i: (0, 0)),
        ],
        out_specs=pl.BlockSpec((PB, 128), lambda i: (i, 0)),
        out_shape=jax.ShapeDtypeStruct((npair * PB, 128), jnp.float32),
    )(table_t, table_t, w)


def _dotp(a_ref, w_ref):
    return lax.dot_general(a_ref[...], w_ref[...],
                           (((0,), (0,)), ((), ())),
                           precision=lax.Precision.DEFAULT,
                           preferred_element_type=jnp.float32)


def _proj_gf_body(tga_ref, tgb_ref, wg_ref, tfa_ref, tfb_ref, wf_ref,
                  outg_ref, outf_ref):
    i = pl.program_id(0)

    @pl.when(i < GN)
    def _():
        outg_ref[...] = jnp.concatenate(
            [_dotp(tga_ref, wg_ref), _dotp(tgb_ref, wg_ref)], axis=-1)

    @pl.when(i == GN)
    def _():
        outf_ref[...] = jnp.concatenate(
            [_dotp(tfa_ref, wf_ref), _dotp(tfb_ref, wf_ref)], axis=-1)


def _project_gf(tg, wg, tf, wf):
    vg = tg.shape[1]
    glast = pl.cdiv(vg, PB) - 1
    return pl.pallas_call(
        _proj_gf_body,
        grid=(GN + 1,),
        in_specs=[
            pl.BlockSpec((DG, PB), lambda i: (0, jnp.minimum(2 * i, glast))),
            pl.BlockSpec((DG, PB),
                         lambda i: (0, jnp.minimum(2 * i + 1, glast))),
            pl.BlockSpec((DG, 64), lambda i: (0, 0)),
            pl.BlockSpec((DF, PB), lambda i: (0, 0)),
            pl.BlockSpec((DF, PB), lambda i: (0, 0)),
            pl.BlockSpec((DF, 64), lambda i: (0, 0)),
        ],
        out_specs=[
            pl.BlockSpec((PB, 128), lambda i: (jnp.minimum(i, GN - 1), 0)),
            pl.BlockSpec((PB, 128), lambda i: (0, 0)),
        ],
        out_shape=[
            jax.ShapeDtypeStruct((GN * PB, 128), jnp.float32),
            jax.ShapeDtypeStruct((PB, 128), jnp.float32),
        ],
    )(tg, tg, wg, tf, tf, wf)


def _sc_body_factory(ntab):
    njob = ntab * NCH
    nbuf = min(NBUF, njob)

    def body(*refs):
        idx_hbms = refs[0:ntab]
        tbls = refs[ntab:2 * ntab]
        outs = refs[2 * ntab:3 * ntab]
        idxs = refs[3 * ntab:4 * ntab]
        bufs = refs[4 * ntab]
        gsem = refs[4 * ntab + 1]
        wsem = refs[4 * ntab + 2]
        wid = lax.axis_index("s") * NC + lax.axis_index("c")
        base = wid * BPW
        crow = wid * NCH
        for ih, iv in zip(idx_hbms, idxs):
            pltpu.sync_copy(ih.at[pl.ds(crow, NCH)], iv)
        jobs = [(tbls[tt], idxs[tt], j)
                for tt in range(ntab) for j in range(NCH)]

        def fire(k):
            tbl, idx, j = jobs[k]
            return pltpu.async_copy(tbl.at[idx.at[j]], bufs.at[k % nbuf], gsem)

        gath = [None] * njob
        wb = [None] * njob
        nprime = min(nbuf, njob)
        for k in range(nprime):
            gath[k] = fire(k)
        for k in range(njob):
            gath[k].wait()
            _, _, j = jobs[k]
            wb[k] = pltpu.async_copy(
                bufs.at[k % nbuf],
                outs[k // NCH].at[pl.ds(base + j * CH, CH)], wsem)
            m = k + nprime
            if m < njob:
                wb[m - nbuf].wait()
                gath[m] = fire(m)
        for k in range(max(0, njob - nbuf), njob):
            wb[k].wait()

    return body


@functools.lru_cache(maxsize=4)
def _make_sc_gather(ntab):
    nbuf = min(NBUF, ntab * NCH)
    return pl.kernel(
        _sc_body_factory(ntab),
        mesh=plsc.VectorSubcoreMesh(core_axis_name="c", subcore_axis_name="s"),
        out_type=[jax.ShapeDtypeStruct((NB, 128), jnp.float32)
                  for _ in range(ntab)],
        scratch_types=(
            [pltpu.VMEM((NCH, CH), jnp.int32) for _ in range(ntab)]
            + [pltpu.VMEM((nbuf, CH, 128), jnp.float32),
               pltpu.SemaphoreType.DMA,
               pltpu.SemaphoreType.DMA]
        ),
    )


def _half(x_t, sel):
    return jnp.where(sel, x_t[64:, :], x_t[:64, :])


def _tc_body(t_ref, g_ref, f_ref, sel_ref, dt_ref, cty_ref, dur_ref,
             bt_ref, wcd_ref, wcc_ref, bc_ref, wn_ref, bn_ref, out_ref):
    sbits = sel_ref[...][None, :]
    t_t = jnp.transpose(t_ref[...])
    g_t = jnp.transpose(g_ref[...])
    f_t = jnp.transpose(f_ref[...])
    tax = (_half(t_t, (sbits & 1) > 0)
           + _half(g_t, (sbits & 2) > 0)
           + _half(f_t, (sbits & 4) > 0)
           + bt_ref[...])
    tax = jnp.maximum(tax, 0.0)
    dt_row = dt_ref[...][None, :]
    cty_row = cty_ref[...][None, :]
    oh_d = (lax.broadcasted_iota(jnp.int32, (16, BS), 0)
            == dt_row).astype(jnp.float32)
    oh_c = (lax.broadcasted_iota(jnp.int32, (256, BS), 0)
            == cty_row).astype(jnp.float32)
    cat = (lax.dot_general(wcd_ref[...], oh_d, (((0,), (0,)), ((), ())),
                           preferred_element_type=jnp.float32)
           + lax.dot_general(wcc_ref[...], oh_c, (((0,), (0,)), ((), ())),
                             preferred_element_type=jnp.float32)
           + bc_ref[...])
    cat = jnp.maximum(cat, 0.0)
    dur_row = dur_ref[...][None, :]
    num = jnp.maximum(
        lax.dot_general(wn_ref[...], dur_row, (((0,), (0,)), ((), ())),
                        preferred_element_type=jnp.float32) + bn_ref[...],
        0.0)
    out_ref[...] = jnp.concatenate([tax, cat, num], axis=0)


PSH = PB.bit_length() - 1


def _pack_idx(idx, v):
    idx = idx.astype(jnp.int32)
    row = (idx >> (PSH + 1)) * PB + (idx & (PB - 1))
    side = (idx >> PSH) & 1
    return row, side


def kernel(taxid, genus, family, device_type, country, duration,
           emb_taxid, emb_genus, emb_family, Wt, bt, Wc, bc, Wn, bn):
    pg, pf = _project_gf(emb_genus.T, Wt[DT:DT + DG],
                         emb_family.T, Wt[DT + DG:])

    row_g, side_g = _pack_idx(genus, pg.shape[0])
    row_f, side_f = _pack_idx(family, pf.shape[0])
    g, f = _make_sc_gather(2)(
        row_g.reshape(NB // CH, CH), row_f.reshape(NB // CH, CH), pg, pf)

    pt = _project2(emb_taxid.T, Wt[0:DT])
    row_t, side_t = _pack_idx(taxid, pt.shape[0])
    selbits = (side_t | (side_g << 1) | (side_f << 2)).astype(jnp.int32)
    t = _make_sc_gather(1)(row_t.reshape(NB // CH, CH), pt)
    if isinstance(t, (list, tuple)):
        t = t[0]

    dt1 = device_type.astype(jnp.int32)
    cty1 = country.astype(jnp.int32)
    bt2 = bt.reshape(DT, 1)
    wcd, wcc = Wc[0:16], Wc[16:]
    bc2 = bc.reshape(16, 1)
    bn2 = bn.reshape(16, 1)

    out_t = pl.pallas_call(
        _tc_body,
        grid=(NB // BS,),
        in_specs=[
            pl.BlockSpec((BS, 128), lambda i: (i, 0)),
            pl.BlockSpec((BS, 128), lambda i: (i, 0)),
            pl.BlockSpec((BS, 128), lambda i: (i, 0)),
            pl.BlockSpec((BS,), lambda i: (i,)),
            pl.BlockSpec((BS,), lambda i: (i,)),
            pl.BlockSpec((BS,), lambda i: (i,)),
            pl.BlockSpec((BS,), lambda i: (i,)),
            pl.BlockSpec((DT, 1), lambda i: (0, 0)),
            pl.BlockSpec((16, 16), lambda i: (0, 0)),
            pl.BlockSpec((256, 16), lambda i: (0, 0)),
            pl.BlockSpec((16, 1), lambda i: (0, 0)),
            pl.BlockSpec((1, 16), lambda i: (0, 0)),
            pl.BlockSpec((16, 1), lambda i: (0, 0)),
        ],
        out_specs=pl.BlockSpec((96, BS), lambda i: (0, i)),
        out_shape=jax.ShapeDtypeStruct((96, NB), jnp.float32),
    )(t, g, f, selbits, dt1, cty1, duration,
      bt2, wcd, wcc, bc2, Wn, bn2)
    return out_t.T


# speedup vs baseline: 1.0405x; 1.0405x over previous
import functools

import jax
import jax.numpy as jnp
from jax import lax
from jax.experimental import pallas as pl
from jax.experimental.pallas import tpu as pltpu
from jax.experimental.pallas import tpu_sc as plsc

NB = 16384
NC, NS = 2, 16
NW = NC * NS
BPW = NB // NW
CH = 128
NCH = BPW // CH
NJOB = 3 * NCH
NBUF = 7

DT, DG, DF = 64, 32, 16
PB = 4096
BS = 2048
GN = 2


def _proj2_body(tva_ref, tvb_ref, w_ref, out_ref):
    wb16 = w_ref[...].astype(jnp.bfloat16)
    pa = lax.dot_general(tva_ref[...].astype(jnp.bfloat16), wb16,
                         (((0,), (0,)), ((), ())),
                         preferred_element_type=jnp.float32)
    pb = lax.dot_general(tvb_ref[...].astype(jnp.bfloat16), wb16,
                         (((0,), (0,)), ((), ())),
                         preferred_element_type=jnp.float32)
    out_ref[...] = jnp.concatenate([pa, pb], axis=-1)


def _project2(table_t, w):
    d, v = table_t.shape
    nblk = pl.cdiv(v, PB)
    npair = pl.cdiv(nblk, 2)
    last = nblk - 1
    return pl.pallas_call(
        _proj2_body,
        grid=(npair,),
        in_specs=[
            pl.BlockSpec((d, PB), lambda i: (0, 2 * i)),
            pl.BlockSpec((d, PB), lambda i: (0, jnp.minimum(2 * i + 1, last))),
            pl.BlockSpec((d, 64), lambda i: (0, 0)),
        ],
        out_specs=pl.BlockSpec((PB, 128), lambda i: (i, 0)),
        out_shape=jax.ShapeDtypeStruct((npair * PB, 128), jnp.float32),
    )(table_t, table_t, w)


def _dotp(a_ref, w_ref):
    return lax.dot_general(a_ref[...], w_ref[...],
                           (((0,), (0,)), ((), ())),
                           precision=lax.Precision.DEFAULT,
                           preferred_element_type=jnp.float32)


def _proj_gf_body(tga_ref, tgb_ref, wg_ref, tfa_ref, tfb_ref, wf_ref,
                  outg_ref, outf_ref):
    i = pl.program_id(0)

    @pl.when(i < GN)
    def _():
        outg_ref[...] = jnp.concatenate(
            [_dotp(tga_ref, wg_ref), _dotp(tgb_ref, wg_ref)], axis=-1)

    @pl.when(i == GN)
    def _():
        outf_ref[...] = jnp.concatenate(
            [_dotp(tfa_ref, wf_ref), _dotp(tfb_ref, wf_ref)], axis=-1)


def _project_gf(tg, wg, tf, wf):
    vg = tg.shape[1]
    glast = pl.cdiv(vg, PB) - 1
    return pl.pallas_call(
        _proj_gf_body,
        grid=(GN + 1,),
        in_specs=[
            pl.BlockSpec((DG, PB), lambda i: (0, jnp.minimum(2 * i, glast))),
            pl.BlockSpec((DG, PB),
                         lambda i: (0, jnp.minimum(2 * i + 1, glast))),
            pl.BlockSpec((DG, 64), lambda i: (0, 0)),
            pl.BlockSpec((DF, PB), lambda i: (0, 0)),
            pl.BlockSpec((DF, PB), lambda i: (0, 0)),
            pl.BlockSpec((DF, 64), lambda i: (0, 0)),
        ],
        out_specs=[
            pl.BlockSpec((PB, 128), lambda i: (jnp.minimum(i, GN - 1), 0)),
            pl.BlockSpec((PB, 128), lambda i: (0, 0)),
        ],
        out_shape=[
            jax.ShapeDtypeStruct((GN * PB, 128), jnp.float32),
            jax.ShapeDtypeStruct((PB, 128), jnp.float32),
        ],
    )(tg, tg, wg, tf, tf, wf)


def _sc_body_factory(ntab):
    njob = ntab * NCH
    nbuf = min(NBUF, njob)

    def body(*refs):
        idx_hbms = refs[0:ntab]
        tbls = refs[ntab:2 * ntab]
        outs = refs[2 * ntab:3 * ntab]
        idxs = refs[3 * ntab:4 * ntab]
        bufs = refs[4 * ntab]
        gsem = refs[4 * ntab + 1]
        wsem = refs[4 * ntab + 2]
        wid = lax.axis_index("s") * NC + lax.axis_index("c")
        base = wid * BPW
        crow = wid * NCH
        for ih, iv in zip(idx_hbms, idxs):
            pltpu.sync_copy(ih.at[pl.ds(crow, NCH)], iv)
        jobs = [(tbls[tt], idxs[tt], j)
                for tt in range(ntab) for j in range(NCH)]

        def fire(k):
            tbl, idx, j = jobs[k]
            return pltpu.async_copy(tbl.at[idx.at[j]], bufs.at[k % nbuf], gsem)

        gath = [None] * njob
        wb = [None] * njob
        nprime = min(nbuf, njob)
        for k in range(nprime):
            gath[k] = fire(k)
        for k in range(njob):
            gath[k].wait()
            _, _, j = jobs[k]
            wb[k] = pltpu.async_copy(
                bufs.at[k % nbuf],
                outs[k // NCH].at[pl.ds(base + j * CH, CH)], wsem)
            m = k + nprime
            if m < njob:
                wb[m - nbuf].wait()
                gath[m] = fire(m)
        for k in range(max(0, njob - nbuf), njob):
            wb[k].wait()

    return body


@functools.lru_cache(maxsize=4)
def _make_sc_gather(ntab):
    nbuf = min(NBUF, ntab * NCH)
    return pl.kernel(
        _sc_body_factory(ntab),
        mesh=plsc.VectorSubcoreMesh(core_axis_name="c", subcore_axis_name="s"),
        out_type=[jax.ShapeDtypeStruct((NB, 128), jnp.float32)
                  for _ in range(ntab)],
        scratch_types=(
            [pltpu.VMEM((NCH, CH), jnp.int32) for _ in range(ntab)]
            + [pltpu.VMEM((nbuf, CH, 128), jnp.float32),
               pltpu.SemaphoreType.DMA,
               pltpu.SemaphoreType.DMA]
        ),
    )


def _half(x_t, sel):
    return jnp.where(sel, x_t[64:, :], x_t[:64, :])


def _tc_body(t_ref, g_ref, f_ref, sel_ref, dt_ref, cty_ref, dur_ref,
             bt_ref, wcd_ref, wcc_ref, bc_ref, wn_ref, bn_ref, out_ref):
    sbits = sel_ref[...][None, :]
    t_t = jnp.transpose(t_ref[...])
    g_t = jnp.transpose(g_ref[...])
    f_t = jnp.transpose(f_ref[...])
    tax = (_half(t_t, (sbits & 1) > 0)
           + _half(g_t, (sbits & 2) > 0)
           + _half(f_t, (sbits & 4) > 0)
           + bt_ref[...])
    tax = jnp.maximum(tax, 0.0)
    dt_row = dt_ref[...][None, :]
    cty_row = cty_ref[...][None, :]
    oh_d = (lax.broadcasted_iota(jnp.int32, (16, BS), 0)
            == dt_row).astype(jnp.float32)
    oh_c = (lax.broadcasted_iota(jnp.int32, (256, BS), 0)
            == cty_row).astype(jnp.float32)
    cat = (lax.dot_general(wcd_ref[...], oh_d, (((0,), (0,)), ((), ())),
                           preferred_element_type=jnp.float32)
           + lax.dot_general(wcc_ref[...], oh_c, (((0,), (0,)), ((), ())),
                             preferred_element_type=jnp.float32)
           + bc_ref[...])
    cat = jnp.maximum(cat, 0.0)
    dur_row = dur_ref[...][None, :]
    num = jnp.maximum(
        lax.dot_general(wn_ref[...], dur_row, (((0,), (0,)), ((), ())),
                        preferred_element_type=jnp.float32) + bn_ref[...],
        0.0)
    out_ref[...] = jnp.concatenate([tax, cat, num], axis=0)


PSH = PB.bit_length() - 1


def _pack_idx(idx, v):
    idx = idx.astype(jnp.int32)
    row = (idx >> (PSH + 1)) * PB + (idx & (PB - 1))
    side = (idx >> PSH) & 1
    return row, side


def kernel(taxid, genus, family, device_type, country, duration,
           emb_taxid, emb_genus, emb_family, Wt, bt, Wc, bc, Wn, bn):
    pg, pf = _project_gf(emb_genus.T, Wt[DT:DT + DG],
                         emb_family.T, Wt[DT + DG:])

    row_g, side_g = _pack_idx(genus, pg.shape[0])
    row_f, side_f = _pack_idx(family, pf.shape[0])
    g, f = _make_sc_gather(2)(
        row_g.reshape(NB // CH, CH), row_f.reshape(NB // CH, CH), pg, pf)

    pt = _project2(emb_taxid.T, Wt[0:DT])
    row_t, side_t = _pack_idx(taxid, pt.shape[0])
    selbits = (side_t | (side_g << 1) | (side_f << 2)).astype(jnp.int32)
    t = _make_sc_gather(1)(row_t.reshape(NB // CH, CH), pt)
    if isinstance(t, (list, tuple)):
        t = t[0]

    dt1 = device_type.astype(jnp.int32)
    cty1 = country.astype(jnp.int32)
    bt2 = bt.reshape(DT, 1)
    wcd, wcc = Wc[0:16], Wc[16:]
    bc2 = bc.reshape(16, 1)
    bn2 = bn.reshape(16, 1)

    out_t = pl.pallas_call(
        _tc_body,
        grid=(NB // BS,),
        in_specs=[
            pl.BlockSpec((BS, 128), lambda i: (i, 0)),
            pl.BlockSpec((BS, 128), lambda i: (i, 0)),
            pl.BlockSpec((BS, 128), lambda i: (i, 0)),
            pl.BlockSpec((BS,), lambda i: (i,)),
            pl.BlockSpec((BS,), lambda i: (i,)),
            pl.BlockSpec((BS,), lambda i: (i,)),
            pl.BlockSpec((BS,), lambda i: (i,)),
            pl.BlockSpec((DT, 1), lambda i: (0, 0)),
            pl.BlockSpec((16, 16), lambda i: (0, 0)),
            pl.BlockSpec((256, 16), lambda i: (0, 0)),
            pl.BlockSpec((16, 1), lambda i: (0, 0)),
            pl.BlockSpec((1, 16), lambda i: (0, 0)),
            pl.BlockSpec((16, 1), lambda i: (0, 0)),
        ],
        out_specs=pl.BlockSpec((96, BS), lambda i: (0, i)),
        out_shape=jax.ShapeDtypeStruct((96, NB), jnp.float32),
    )(t, g, f, selbits, dt1, cty1, duration,
      bt2, wcd, wcc, bc2, Wn, bn2)
    return out_t.T
